# baseline (speedup 1.0000x reference)
import functools

import jax
import jax.numpy as jnp
from jax import lax
from jax.experimental import pallas as pl
from jax.experimental.pallas import tpu as pltpu
from jax.experimental.pallas import tpu_sc as plsc

N = 10000
D = 128
E = 320000

NPAD = 10240
NC = 2
NS = 16
CB = 128
DUMMY = 10016

TOT_CHUNKS = 2560
EPAD_AGG = TOT_CHUNKS * CB
CH0 = 32
CH1 = (TOT_CHUNKS - NS * CH0) // NS
GROUP = 32

DEG_CHUNKS = 160
EW_DEG = DEG_CHUNKS * CB
EPAD_DEG = NS * EW_DEG

ROWS_PER_TILE = NPAD // NS


def _fill_const(ref, nrows, value):
    ncols = ref.shape[1]

    def body(i, _):
        for k in range(ncols // 16):
            ref[i, pl.ds(k * 16, 16)] = jnp.full((16,), value, jnp.float32)
        return 0

    lax.fori_loop(0, nrows, body, 0)


@functools.cache
def _sc_degrees_kernel():
    mesh = plsc.VectorSubcoreMesh(core_axis_name="c", subcore_axis_name="s")

    def body(idx_hbm, out_hbm, idx, ones, counts, sem):
        c = lax.axis_index("c")
        s = lax.axis_index("s")

        _fill_const(ones, CB, 0.0)
        def zcp(t, _):
            pltpu.sync_copy(ones, counts.at[pl.ds(s * ROWS_PER_TILE + t * CB, CB)])
            return 0
        lax.fori_loop(0, ROWS_PER_TILE // CB, zcp, 0)
        _fill_const(ones, CB, 1.0)

        pltpu.sync_copy(
            idx_hbm.at[pl.ds(c * (EPAD_DEG // CB) + s * DEG_CHUNKS, DEG_CHUNKS)], idx)
        plsc.subcore_barrier()

        def chunk(j, _):
            pltpu.sync_copy(ones, counts.at[idx.at[j]], add=True)
            return 0
        lax.fori_loop(0, DEG_CHUNKS, chunk, 0)
        plsc.subcore_barrier()

        def ocp(t, _):
            r = s * ROWS_PER_TILE + t * CB
            pltpu.sync_copy(counts.at[pl.ds(r, CB)], out_hbm.at[c, pl.ds(r, CB)])
            return 0
        lax.fori_loop(0, ROWS_PER_TILE // CB, ocp, 0)

    return pl.kernel(
        body,
        out_type=jax.ShapeDtypeStruct((NC, NPAD, D), jnp.float32),
        mesh=mesh,
        scratch_types=[
            pltpu.VMEM((DEG_CHUNKS, CB), jnp.int32),
            pltpu.VMEM((CB, D), jnp.float32),
            pltpu.VMEM_SHARED((NPAD, D), jnp.float32),
            pltpu.SemaphoreType.DMA,
        ],
    )


@functools.cache
def _sc_aggregate_kernel():
    mesh = plsc.VectorSubcoreMesh(core_axis_name="c", subcore_axis_name="s")

    def body(hs_hbm, src_hbm, dst_hbm, out_hbm, sidx, didx, rows_a, rows_b, acc,
             sem_a, sem_b):
        c = lax.axis_index("c")
        s = lax.axis_index("s")

        _fill_const(rows_a, CB, 0.0)
        def zcp(t, _):
            pltpu.sync_copy(rows_a, acc.at[pl.ds(s * ROWS_PER_TILE + t * CB, CB)])
            return 0
        lax.fori_loop(0, ROWS_PER_TILE // CB, zcp, 0)

        plsc.subcore_barrier()

        n_groups = jnp.where(c == 0, CH0 // GROUP, CH1 // GROUP)
        base = jnp.where(c == 0, s * CH0, NS * CH0 + s * CH1)

        def group_body(g, _):
            gb = base + g * GROUP
            pltpu.sync_copy(src_hbm.at[pl.ds(gb, GROUP)], sidx)
            pltpu.sync_copy(dst_hbm.at[pl.ds(gb, GROUP)], didx)

            pltpu.async_copy(hs_hbm.at[sidx.at[0]], rows_a, sem_a)
            pltpu.async_copy(hs_hbm.at[sidx.at[1]], rows_b, sem_b)

            def step(t, _):
                ja = 2 * t
                jb = ja + 1
                pltpu.make_async_copy(hs_hbm.at[sidx.at[ja]], rows_a, sem_a).wait()
                pltpu.sync_copy(rows_a, acc.at[didx.at[ja]], add=True)
                pltpu.async_copy(hs_hbm.at[sidx.at[ja + 2]], rows_a, sem_a)
                pltpu.make_async_copy(hs_hbm.at[sidx.at[jb]], rows_b, sem_b).wait()
                pltpu.sync_copy(rows_b, acc.at[didx.at[jb]], add=True)
                pltpu.async_copy(hs_hbm.at[sidx.at[jb + 2]], rows_b, sem_b)
                return 0
            lax.fori_loop(0, GROUP // 2 - 1, step, 0)

            for j in (GROUP - 2, GROUP - 1):
                buf, sem = (rows_a, sem_a) if j % 2 == 0 else (rows_b, sem_b)
                pltpu.make_async_copy(hs_hbm.at[sidx.at[j]], buf, sem).wait()
                pltpu.sync_copy(buf, acc.at[didx.at[j]], add=True)
            return 0

        lax.fori_loop(0, n_groups, group_body, 0)
        plsc.subcore_barrier()

        def ocp(t, _):
            r = s * ROWS_PER_TILE + t * CB
            pltpu.sync_copy(acc.at[pl.ds(r, CB)], out_hbm.at[c, pl.ds(r, CB)])
            return 0
        lax.fori_loop(0, ROWS_PER_TILE // CB, ocp, 0)

    return pl.kernel(
        body,
        out_type=jax.ShapeDtypeStruct((NC, NPAD, D), jnp.float32),
        mesh=mesh,
        scratch_types=[
            pltpu.VMEM((GROUP, CB), jnp.int32),
            pltpu.VMEM((GROUP, CB), jnp.int32),
            pltpu.VMEM((CB, D), jnp.float32),
            pltpu.VMEM((CB, D), jnp.float32),
            pltpu.VMEM_SHARED((NPAD, D), jnp.float32),
            pltpu.SemaphoreType.DMA,
            pltpu.SemaphoreType.DMA,
        ],
    )


BLK = 1024


def _norm(deg_blk):
    return lax.rsqrt(jnp.maximum(deg_blk, 1.0))


def _tc_entry(x, w, deg_out):
    def body(x_ref, w_ref, d_ref, o_ref):
        h = jnp.dot(x_ref[...], w_ref[...],
                    preferred_element_type=jnp.float32,
                    precision=lax.Precision.HIGHEST)
        o_ref[...] = h * _norm(d_ref[...])

    return pl.pallas_call(
        body,
        grid=(NPAD // BLK,),
        in_specs=[
            pl.BlockSpec((BLK, D), lambda i: (i, 0)),
            pl.BlockSpec((D, D), lambda i: (0, 0)),
            pl.BlockSpec((BLK, 1), lambda i: (i, 0)),
        ],
        out_specs=pl.BlockSpec((BLK, D), lambda i: (i, 0)),
        out_shape=jax.ShapeDtypeStruct((NPAD, D), jnp.float32),
    )(x, w, deg_out)


def _tc_mid(agg, deg_in, b, w, deg_out):
    def body(a_ref, di_ref, b_ref, w_ref, do_ref, o_ref):
        total = a_ref[0] + a_ref[1]
        h = jnp.maximum(total * _norm(di_ref[...]) + b_ref[...], 0.0)
        hs = jnp.dot(h, w_ref[...],
                     preferred_element_type=jnp.float32,
                     precision=lax.Precision.HIGHEST)
        o_ref[...] = hs * _norm(do_ref[...])

    return pl.pallas_call(
        body,
        grid=(NPAD // BLK,),
        in_specs=[
            pl.BlockSpec((NC, BLK, D), lambda i: (0, i, 0)),
            pl.BlockSpec((BLK, 1), lambda i: (i, 0)),
            pl.BlockSpec((1, D), lambda i: (0, 0)),
            pl.BlockSpec((D, D), lambda i: (0, 0)),
            pl.BlockSpec((BLK, 1), lambda i: (i, 0)),
        ],
        out_specs=pl.BlockSpec((BLK, D), lambda i: (i, 0)),
        out_shape=jax.ShapeDtypeStruct((NPAD, D), jnp.float32),
    )(agg, deg_in, b, w, deg_out)


def _tc_final(agg, deg_in, b):
    def body(a_ref, di_ref, b_ref, o_ref):
        total = a_ref[0] + a_ref[1]
        o_ref[...] = total * _norm(di_ref[...]) + b_ref[...]

    return pl.pallas_call(
        body,
        grid=(NPAD // BLK,),
        in_specs=[
            pl.BlockSpec((NC, BLK, D), lambda i: (0, i, 0)),
            pl.BlockSpec((BLK, 1), lambda i: (i, 0)),
            pl.BlockSpec((1, D), lambda i: (0, 0)),
        ],
        out_specs=pl.BlockSpec((BLK, D), lambda i: (i, 0)),
        out_shape=jax.ShapeDtypeStruct((NPAD, D), jnp.float32),
    )(agg, deg_in, b)


def kernel(x, edge_index, W0, b0, W1, b1, W2, b2):
    src = edge_index[0]
    dst = edge_index[1]

    pad_a = EPAD_AGG - E
    src_a = jnp.concatenate([src, jnp.zeros((pad_a,), jnp.int32)]).reshape(-1, CB)
    dst_a = jnp.concatenate([dst, jnp.full((pad_a,), DUMMY, jnp.int32)]).reshape(-1, CB)
    src_d = jnp.concatenate([src, jnp.full((EPAD_DEG - E,), DUMMY, jnp.int32)]).reshape(-1, CB)
    idx_d = jnp.concatenate([src_d, dst_a])

    x_pad = jnp.pad(x, ((0, NPAD - N), (0, 0)))
    b0_2d = b0.reshape(1, D)
    b1_2d = b1.reshape(1, D)
    b2_2d = b2.reshape(1, D)

    deg2 = _sc_degrees_kernel()(idx_d)
    deg_out = deg2[0, :, :1]
    deg_in = deg2[1, :, :1]

    agg_fn = _sc_aggregate_kernel()

    hs = _tc_entry(x_pad, W0, deg_out)
    agg = agg_fn(hs, src_a, dst_a)
    hs = _tc_mid(agg, deg_in, b0_2d, W1, deg_out)
    agg = agg_fn(hs, src_a, dst_a)
    hs = _tc_mid(agg, deg_in, b1_2d, W2, deg_out)
    agg = agg_fn(hs, src_a, dst_a)
    out = _tc_final(agg, deg_in, b2_2d)
    return out[:N]


# speedup vs baseline: 1.0894x; 1.0894x over previous
import functools

import jax
import jax.numpy as jnp
from jax import lax
from jax.experimental import pallas as pl
from jax.experimental.pallas import tpu as pltpu
from jax.experimental.pallas import tpu_sc as plsc

N = 10000
D = 128
E = 320000

NPAD = 10240
NC = 2
NS = 16
CB = 128
DUMMY = 10016

TOT_CHUNKS = 2560
EPAD_AGG = TOT_CHUNKS * CB
CH0 = 128
CH1 = (TOT_CHUNKS - NS * CH0) // NS
GROUP = 32

DEG_CHUNKS = 160
EW_DEG = DEG_CHUNKS * CB
EPAD_DEG = NS * EW_DEG

ROWS_PER_TILE = NPAD // NS


def _fill_const(ref, nrows, value):
    ncols = ref.shape[1]

    def body(i, _):
        for k in range(ncols // 16):
            ref[i, pl.ds(k * 16, 16)] = jnp.full((16,), value, jnp.float32)
        return 0

    lax.fori_loop(0, nrows, body, 0)


@functools.cache
def _sc_degrees_kernel():
    mesh = plsc.VectorSubcoreMesh(core_axis_name="c", subcore_axis_name="s")

    def body(idx_hbm, out_hbm, idx, ones, counts, sem):
        c = lax.axis_index("c")
        s = lax.axis_index("s")

        _fill_const(ones, CB, 0.0)
        def zcp(t, _):
            pltpu.sync_copy(ones, counts.at[pl.ds(s * ROWS_PER_TILE + t * CB, CB)])
            return 0
        lax.fori_loop(0, ROWS_PER_TILE // CB, zcp, 0)
        _fill_const(ones, CB, 1.0)

        pltpu.sync_copy(
            idx_hbm.at[pl.ds(c * (EPAD_DEG // CB) + s * DEG_CHUNKS, DEG_CHUNKS)], idx)
        plsc.subcore_barrier()

        def chunk(j, _):
            pltpu.sync_copy(ones, counts.at[idx.at[j]], add=True)
            return 0
        lax.fori_loop(0, DEG_CHUNKS, chunk, 0)
        plsc.subcore_barrier()

        def ocp(t, _):
            r = s * ROWS_PER_TILE + t * CB
            pltpu.sync_copy(counts.at[pl.ds(r, CB)], out_hbm.at[c, pl.ds(r, CB)])
            return 0
        lax.fori_loop(0, ROWS_PER_TILE // CB, ocp, 0)

    return pl.kernel(
        body,
        out_type=jax.ShapeDtypeStruct((NC, NPAD, D), jnp.float32),
        mesh=mesh,
        scratch_types=[
            pltpu.VMEM((DEG_CHUNKS, CB), jnp.int32),
            pltpu.VMEM((CB, D), jnp.float32),
            pltpu.VMEM_SHARED((NPAD, D), jnp.float32),
            pltpu.SemaphoreType.DMA,
        ],
    )


@functools.cache
def _sc_aggregate_kernel():
    mesh = plsc.VectorSubcoreMesh(core_axis_name="c", subcore_axis_name="s")

    def body(hs_hbm, src_hbm, dst_hbm, out_hbm, sidx, didx, rows_a, rows_b, acc,
             sem_a, sem_b):
        c = lax.axis_index("c")
        s = lax.axis_index("s")

        _fill_const(rows_a, CB, 0.0)
        def zcp(t, _):
            pltpu.sync_copy(rows_a, acc.at[pl.ds(s * ROWS_PER_TILE + t * CB, CB)])
            return 0
        lax.fori_loop(0, ROWS_PER_TILE // CB, zcp, 0)

        plsc.subcore_barrier()

        n_groups = jnp.where(c == 0, CH0 // GROUP, CH1 // GROUP)
        base = jnp.where(c == 0, s * CH0, NS * CH0 + s * CH1)

        def group_body(g, _):
            gb = base + g * GROUP
            pltpu.sync_copy(src_hbm.at[pl.ds(gb, GROUP)], sidx)
            pltpu.sync_copy(dst_hbm.at[pl.ds(gb, GROUP)], didx)

            pltpu.async_copy(hs_hbm.at[sidx.at[0]], rows_a, sem_a)
            pltpu.async_copy(hs_hbm.at[sidx.at[1]], rows_b, sem_b)

            def step(t, _):
                ja = 2 * t
                jb = ja + 1
                pltpu.make_async_copy(hs_hbm.at[sidx.at[ja]], rows_a, sem_a).wait()
                pltpu.sync_copy(rows_a, acc.at[didx.at[ja]], add=True)
                pltpu.async_copy(hs_hbm.at[sidx.at[ja + 2]], rows_a, sem_a)
                pltpu.make_async_copy(hs_hbm.at[sidx.at[jb]], rows_b, sem_b).wait()
                pltpu.sync_copy(rows_b, acc.at[didx.at[jb]], add=True)
                pltpu.async_copy(hs_hbm.at[sidx.at[jb + 2]], rows_b, sem_b)
                return 0
            lax.fori_loop(0, GROUP // 2 - 1, step, 0)

            for j in (GROUP - 2, GROUP - 1):
                buf, sem = (rows_a, sem_a) if j % 2 == 0 else (rows_b, sem_b)
                pltpu.make_async_copy(hs_hbm.at[sidx.at[j]], buf, sem).wait()
                pltpu.sync_copy(buf, acc.at[didx.at[j]], add=True)
            return 0

        lax.fori_loop(0, n_groups, group_body, 0)
        plsc.subcore_barrier()

        def ocp(t, _):
            r = s * ROWS_PER_TILE + t * CB
            pltpu.sync_copy(acc.at[pl.ds(r, CB)], out_hbm.at[c, pl.ds(r, CB)])
            return 0
        lax.fori_loop(0, ROWS_PER_TILE // CB, ocp, 0)

    return pl.kernel(
        body,
        out_type=jax.ShapeDtypeStruct((NC, NPAD, D), jnp.float32),
        mesh=mesh,
        scratch_types=[
            pltpu.VMEM((GROUP, CB), jnp.int32),
            pltpu.VMEM((GROUP, CB), jnp.int32),
            pltpu.VMEM((CB, D), jnp.float32),
            pltpu.VMEM((CB, D), jnp.float32),
            pltpu.VMEM_SHARED((NPAD, D), jnp.float32),
            pltpu.SemaphoreType.DMA,
            pltpu.SemaphoreType.DMA,
        ],
    )


BLK = 1024


def _norm(deg_blk):
    return lax.rsqrt(jnp.maximum(deg_blk, 1.0))


def _tc_entry(x, w, deg_out):
    def body(x_ref, w_ref, d_ref, o_ref):
        h = jnp.dot(x_ref[...], w_ref[...],
                    preferred_element_type=jnp.float32,
                    precision=lax.Precision.HIGHEST)
        o_ref[...] = h * _norm(d_ref[...])

    return pl.pallas_call(
        body,
        grid=(NPAD // BLK,),
        in_specs=[
            pl.BlockSpec((BLK, D), lambda i: (i, 0)),
            pl.BlockSpec((D, D), lambda i: (0, 0)),
            pl.BlockSpec((BLK, 1), lambda i: (i, 0)),
        ],
        out_specs=pl.BlockSpec((BLK, D), lambda i: (i, 0)),
        out_shape=jax.ShapeDtypeStruct((NPAD, D), jnp.float32),
    )(x, w, deg_out)


def _tc_mid(agg, deg_in, b, w, deg_out):
    def body(a_ref, di_ref, b_ref, w_ref, do_ref, o_ref):
        total = a_ref[0] + a_ref[1]
        h = jnp.maximum(total * _norm(di_ref[...]) + b_ref[...], 0.0)
        hs = jnp.dot(h, w_ref[...],
                     preferred_element_type=jnp.float32,
                     precision=lax.Precision.HIGHEST)
        o_ref[...] = hs * _norm(do_ref[...])

    return pl.pallas_call(
        body,
        grid=(NPAD // BLK,),
        in_specs=[
            pl.BlockSpec((NC, BLK, D), lambda i: (0, i, 0)),
            pl.BlockSpec((BLK, 1), lambda i: (i, 0)),
            pl.BlockSpec((1, D), lambda i: (0, 0)),
            pl.BlockSpec((D, D), lambda i: (0, 0)),
            pl.BlockSpec((BLK, 1), lambda i: (i, 0)),
        ],
        out_specs=pl.BlockSpec((BLK, D), lambda i: (i, 0)),
        out_shape=jax.ShapeDtypeStruct((NPAD, D), jnp.float32),
    )(agg, deg_in, b, w, deg_out)


def _tc_final(agg, deg_in, b):
    def body(a_ref, di_ref, b_ref, o_ref):
        total = a_ref[0] + a_ref[1]
        o_ref[...] = total * _norm(di_ref[...]) + b_ref[...]

    return pl.pallas_call(
        body,
        grid=(NPAD // BLK,),
        in_specs=[
            pl.BlockSpec((NC, BLK, D), lambda i: (0, i, 0)),
            pl.BlockSpec((BLK, 1), lambda i: (i, 0)),
            pl.BlockSpec((1, D), lambda i: (0, 0)),
        ],
        out_specs=pl.BlockSpec((BLK, D), lambda i: (i, 0)),
        out_shape=jax.ShapeDtypeStruct((NPAD, D), jnp.float32),
    )(agg, deg_in, b)


def kernel(x, edge_index, W0, b0, W1, b1, W2, b2):
    src = edge_index[0]
    dst = edge_index[1]

    pad_a = EPAD_AGG - E
    src_a = jnp.concatenate([src, jnp.zeros((pad_a,), jnp.int32)]).reshape(-1, CB)
    dst_a = jnp.concatenate([dst, jnp.full((pad_a,), DUMMY, jnp.int32)]).reshape(-1, CB)
    src_d = jnp.concatenate([src, jnp.full((EPAD_DEG - E,), DUMMY, jnp.int32)]).reshape(-1, CB)
    idx_d = jnp.concatenate([src_d, dst_a])

    x_pad = jnp.pad(x, ((0, NPAD - N), (0, 0)))
    b0_2d = b0.reshape(1, D)
    b1_2d = b1.reshape(1, D)
    b2_2d = b2.reshape(1, D)

    deg2 = _sc_degrees_kernel()(idx_d)
    deg_out = deg2[0, :, :1]
    deg_in = deg2[1, :, :1]

    agg_fn = _sc_aggregate_kernel()

    hs = _tc_entry(x_pad, W0, deg_out)
    agg = agg_fn(hs, src_a, dst_a)
    hs = _tc_mid(agg, deg_in, b0_2d, W1, deg_out)
    agg = agg_fn(hs, src_a, dst_a)
    hs = _tc_mid(agg, deg_in, b1_2d, W2, deg_out)
    agg = agg_fn(hs, src_a, dst_a)
    out = _tc_final(agg, deg_in, b2_2d)
    return out[:N]
